# Initial kernel scaffold; baseline (speedup 1.0000x reference)
import jax
import jax.numpy as jnp
from jax.experimental import pallas as pl


def kernel(input, target, class_num):
    raise NotImplementedError("write your pallas kernel here")


# speedup vs baseline: 1.3824x; 1.3824x over previous
import functools

import jax
import jax.numpy as jnp
from jax import lax
from jax.experimental import pallas as pl
from jax.experimental.pallas import tpu as pltpu
from jax.experimental.pallas import tpu_sc as plsc

NC = 2
NS = 16
NW = NC * NS
L = 16
ROW = 32
P = 2048


def _sc_partial_hist(inp3, tgt2, C):
    B, _, HW = inp3.shape
    ppw = HW // NW
    nchunk = ppw // P
    HB = C * ROW

    mesh = plsc.VectorSubcoreMesh(core_axis_name="c", subcore_axis_name="s")

    @functools.partial(
        pl.kernel,
        mesh=mesh,
        compiler_params=pltpu.CompilerParams(needs_layout_passes=False),
        out_type=jax.ShapeDtypeStruct((NW, L * HB), jnp.int32),
        scratch_types=[
            pltpu.VMEM((C, P), jnp.float32),
            pltpu.VMEM((P,), jnp.int32),
            pltpu.VMEM((L * HB,), jnp.int32),
        ],
    )
    def k(in_hbm, tg_hbm, out_hbm, xbuf, tbuf, hist):
        wid = lax.axis_index("s") * NC + lax.axis_index("c")
        lane_off = lax.broadcasted_iota(jnp.int32, (L,), 0) * HB
        zeros = jnp.zeros((L,), jnp.int32)
        ones = jnp.ones((L,), jnp.int32)

        def zero_body(i, _):
            hist[pl.ds(i * L, L)] = zeros
            return 0

        lax.fori_loop(0, (L * HB) // L, zero_body, 0)

        def chunk_body(ci, _):
            b = ci // nchunk
            off = wid * ppw + (ci % nchunk) * P
            pltpu.sync_copy(in_hbm.at[b, :, pl.ds(off, P)], xbuf)
            pltpu.sync_copy(tg_hbm.at[b, pl.ds(off, P)], tbuf)

            def grp(i, _):
                base = i * L
                m = xbuf[0, pl.ds(base, L)]
                a = zeros
                for c in range(1, C):
                    v = xbuf[c, pl.ds(base, L)]
                    gt = v > m
                    m = jnp.where(gt, v, m)
                    a = jnp.where(gt, c, a)
                t = tbuf[pl.ds(base, L)]
                addr = lane_off + t * ROW + a
                plsc.addupdate_scatter(hist, [addr], ones)
                return 0

            lax.fori_loop(0, P // L, grp, 0)
            return 0

        lax.fori_loop(0, B * nchunk, chunk_body, 0)
        pltpu.sync_copy(hist, out_hbm.at[wid])

    return k(inp3, tgt2)


def _merge(parts, C):
    def body(x_ref, o_ref):
        o_ref[...] = jnp.sum(x_ref[...], axis=0)[:, :C]

    return pl.pallas_call(
        body,
        out_shape=jax.ShapeDtypeStruct((C, C), jnp.int32),
    )(parts)


def kernel(input, target, class_num):
    B, C, H, W = input.shape
    HW = H * W
    inp3 = input.reshape(B, C, HW)
    tgt2 = target.reshape(B, HW)
    parts = _sc_partial_hist(inp3, tgt2, C)
    parts = parts.reshape(NW * L, C, ROW)
    return _merge(parts, C)


# speedup vs baseline: 1.8951x; 1.3708x over previous
import functools

import jax
import jax.numpy as jnp
from jax import lax
from jax.experimental import pallas as pl
from jax.experimental.pallas import tpu as pltpu
from jax.experimental.pallas import tpu_sc as plsc

NC = 2
NS = 16
NW = NC * NS
L = 16
ROW = 32
P = 2048


def _sc_partial_hist(inp3, tgt2, C):
    B, _, HW = inp3.shape
    ppw = HW // NW
    nchunk = ppw // P
    total = B * nchunk
    HB = C * ROW

    mesh = plsc.VectorSubcoreMesh(core_axis_name="c", subcore_axis_name="s")

    @functools.partial(
        pl.kernel,
        mesh=mesh,
        compiler_params=pltpu.CompilerParams(needs_layout_passes=False),
        out_type=jax.ShapeDtypeStruct((NW, L * HB), jnp.int32),
        scratch_types=[
            pltpu.VMEM((2, C, P), jnp.float32),
            pltpu.VMEM((2, P), jnp.int32),
            pltpu.VMEM((L * HB,), jnp.int32),
            pltpu.SemaphoreType.DMA,
            pltpu.SemaphoreType.DMA,
        ],
    )
    def k(in_hbm, tg_hbm, out_hbm, xbufs, tbufs, hist, sem0, sem1):
        wid = lax.axis_index("s") * NC + lax.axis_index("c")
        sems = (sem0, sem1)
        lane_off = lax.broadcasted_iota(jnp.int32, (L,), 0) * HB
        zeros = jnp.zeros((L,), jnp.int32)
        ones = jnp.ones((L,), jnp.int32)

        def zero_body(i, _):
            hist[pl.ds(i * L, L)] = zeros
            return 0

        lax.fori_loop(0, (L * HB) // L, zero_body, 0)

        def issue(ci, slot):
            b = ci // nchunk
            off = wid * ppw + (ci % nchunk) * P
            pltpu.async_copy(in_hbm.at[b, :, pl.ds(off, P)], xbufs.at[slot],
                             sems[slot])
            pltpu.async_copy(tg_hbm.at[b, pl.ds(off, P)], tbufs.at[slot],
                             sems[slot])

        def wait(slot):
            pltpu.make_async_copy(in_hbm.at[0, :, pl.ds(0, P)],
                                  xbufs.at[slot], sems[slot]).wait()
            pltpu.make_async_copy(tg_hbm.at[0, pl.ds(0, P)],
                                  tbufs.at[slot], sems[slot]).wait()

        def group(slot, base):
            items = [(xbufs[slot, c, pl.ds(base, L)], c) for c in range(C)]
            while len(items) > 1:
                nxt = []
                for j in range(0, len(items) - 1, 2):
                    pm, pa = items[j]
                    qm, qa = items[j + 1]
                    gt = qm > pm
                    nxt.append((jnp.where(gt, qm, pm), jnp.where(gt, qa, pa)))
                if len(items) % 2:
                    nxt.append(items[-1])
                items = nxt
            a = items[0][1]
            t = tbufs[slot, pl.ds(base, L)]
            addr = lane_off + t * ROW + a
            plsc.addupdate_scatter(hist, [addr], ones)

        def compute(slot):
            def grp(i, _):
                group(slot, i * (2 * L))
                group(slot, i * (2 * L) + L)
                return 0

            lax.fori_loop(0, P // (2 * L), grp, 0)

        issue(0, 0)
        issue(1, 1)

        def pair_body(cp, _):
            ci = cp * 2
            wait(0)
            compute(0)

            @pl.when(ci + 2 < total)
            def _():
                issue(ci + 2, 0)

            wait(1)
            compute(1)

            @pl.when(ci + 3 < total)
            def _():
                issue(ci + 3, 1)

            return 0

        lax.fori_loop(0, total // 2, pair_body, 0)
        pltpu.sync_copy(hist, out_hbm.at[wid])

    return k(inp3, tgt2)


def _merge(parts, C):
    def body(x_ref, o_ref):
        o_ref[...] = jnp.sum(x_ref[...], axis=0)[:, :C]

    return pl.pallas_call(
        body,
        out_shape=jax.ShapeDtypeStruct((C, C), jnp.int32),
    )(parts)


def kernel(input, target, class_num):
    B, C, H, W = input.shape
    HW = H * W
    inp3 = input.reshape(B, C, HW)
    tgt2 = target.reshape(B, HW)
    parts = _sc_partial_hist(inp3, tgt2, C)
    parts = parts.reshape(NW * L, C, ROW)
    return _merge(parts, C)


# speedup vs baseline: 5.0293x; 2.6539x over previous
import functools

import jax
import jax.numpy as jnp
from jax import lax
from jax.experimental import pallas as pl
from jax.experimental.pallas import tpu as pltpu
from jax.experimental.pallas import tpu_sc as plsc

NC = 2
NS = 16
NW = NC * NS
L = 16
ROW = 32
P = 2048


def _sc_partial_hist(inp, tgt, C):
    B, _, H, W = inp.shape
    HW = H * W
    ppw = HW // NW
    nchunk = ppw // P
    total = B * nchunk
    RPC = P // W
    rpt = ppw // W

    mesh = plsc.VectorSubcoreMesh(core_axis_name="c", subcore_axis_name="s")

    @functools.partial(
        pl.kernel,
        mesh=mesh,
        compiler_params=pltpu.CompilerParams(needs_layout_passes=False),
        out_type=jax.ShapeDtypeStruct((NW, L * C * ROW), jnp.int32),
        scratch_types=[
            pltpu.VMEM((2, C, RPC, W), jnp.float32),
            pltpu.VMEM((2, RPC, W), jnp.int32),
            pltpu.VMEM((L * C * ROW,), jnp.int32),
            pltpu.SemaphoreType.DMA,
            pltpu.SemaphoreType.DMA,
        ],
    )
    def k(in_hbm, tg_hbm, out_hbm, xbufs, tbufs, hist, sem0, sem1):
        wid = lax.axis_index("s") * NC + lax.axis_index("c")
        sems = (sem0, sem1)
        HB = C * ROW
        lane_off = lax.broadcasted_iota(jnp.int32, (L,), 0) * HB
        zeros = jnp.zeros((L,), jnp.int32)
        ones = jnp.ones((L,), jnp.int32)

        def zero_body(i, _):
            hist[pl.ds(i * L, L)] = zeros
            return 0

        lax.fori_loop(0, (L * HB) // L, zero_body, 0)

        def issue(ci, slot):
            b = ci // nchunk
            r0 = wid * rpt + (ci % nchunk) * RPC
            pltpu.async_copy(in_hbm.at[b, :, pl.ds(r0, RPC), :],
                             xbufs.at[slot], sems[slot])
            pltpu.async_copy(tg_hbm.at[b, pl.ds(r0, RPC), :],
                             tbufs.at[slot], sems[slot])

        def wait(slot):
            pltpu.make_async_copy(in_hbm.at[0, :, pl.ds(0, RPC), :],
                                  xbufs.at[slot], sems[slot]).wait()
            pltpu.make_async_copy(tg_hbm.at[0, pl.ds(0, RPC), :],
                                  tbufs.at[slot], sems[slot]).wait()

        def group(slot, r, col):
            items = [(xbufs[slot, c, r, pl.ds(col, L)], c) for c in range(C)]
            while len(items) > 1:
                nxt = []
                for j in range(0, len(items) - 1, 2):
                    pm, pa = items[j]
                    qm, qa = items[j + 1]
                    gt = qm > pm
                    nxt.append((jnp.where(gt, qm, pm), jnp.where(gt, qa, pa)))
                if len(items) % 2:
                    nxt.append(items[-1])
                items = nxt
            a = items[0][1]
            t = tbufs[slot, r, pl.ds(col, L)]
            addr = lane_off + t * ROW + a
            plsc.addupdate_scatter(hist, [addr], ones)

        def compute(slot):
            for r in range(RPC):
                def grp(i, _):
                    group(slot, r, i * (2 * L))
                    group(slot, r, i * (2 * L) + L)
                    return 0

                lax.fori_loop(0, W // (2 * L), grp, 0)

        issue(0, 0)
        issue(1, 1)

        def pair_body(cp, _):
            ci = cp * 2
            wait(0)
            compute(0)

            @pl.when(ci + 2 < total)
            def _():
                issue(ci + 2, 0)

            wait(1)
            compute(1)

            @pl.when(ci + 3 < total)
            def _():
                issue(ci + 3, 1)

            return 0

        lax.fori_loop(0, total // 2, pair_body, 0)
        pltpu.sync_copy(hist, out_hbm.at[wid])

    return k(inp, tgt)


def _merge(parts, C):
    def body(x_ref, o_ref):
        o_ref[...] = jnp.sum(x_ref[...], axis=(0, 1))[:, :C]

    return pl.pallas_call(
        body,
        out_shape=jax.ShapeDtypeStruct((C, C), jnp.int32),
    )(parts)


def kernel(input, target, class_num):
    C = input.shape[1]
    parts = _sc_partial_hist(input, target, C)
    parts = parts.reshape(NW, L, C, ROW)
    return _merge(parts, C)


# speedup vs baseline: 5.4677x; 1.0872x over previous
import functools

import jax
import jax.numpy as jnp
from jax import lax
from jax.experimental import pallas as pl
from jax.experimental.pallas import tpu as pltpu
from jax.experimental.pallas import tpu_sc as plsc

NC = 2
NS = 16
NW = NC * NS
L = 16
ROW = 32
P = 2048


def _sc_partial_hist(inp, tgt, C):
    B, _, H, W = inp.shape
    HW = H * W
    ppw = HW // NW
    nchunk = ppw // P
    total = B * nchunk
    RPC = P // W
    rpt = ppw // W

    mesh = plsc.VectorSubcoreMesh(core_axis_name="c", subcore_axis_name="s")

    @functools.partial(
        pl.kernel,
        mesh=mesh,
        compiler_params=pltpu.CompilerParams(needs_layout_passes=False),
        out_type=jax.ShapeDtypeStruct((NW, L * C * ROW), jnp.int32),
        scratch_types=[
            pltpu.VMEM((2, C, RPC, W), jnp.float32),
            pltpu.VMEM((2, RPC, W), jnp.int32),
            pltpu.VMEM((L * C * ROW,), jnp.int32),
            pltpu.SemaphoreType.DMA,
            pltpu.SemaphoreType.DMA,
        ],
    )
    def k(in_hbm, tg_hbm, out_hbm, xbufs, tbufs, hist, sem0, sem1):
        wid = lax.axis_index("s") * NC + lax.axis_index("c")
        sems = (sem0, sem1)
        HB = C * ROW
        lane_off = lax.broadcasted_iota(jnp.int32, (L,), 0) * HB
        zeros = jnp.zeros((L,), jnp.int32)
        ones = jnp.ones((L,), jnp.int32)

        def zero_body(i, _):
            hist[pl.ds(i * L, L)] = zeros
            return 0

        lax.fori_loop(0, (L * HB) // L, zero_body, 0)

        def issue(ci, slot):
            b = ci // nchunk
            r0 = wid * rpt + (ci % nchunk) * RPC
            pltpu.async_copy(in_hbm.at[b, :, pl.ds(r0, RPC), :],
                             xbufs.at[slot], sems[slot])
            pltpu.async_copy(tg_hbm.at[b, pl.ds(r0, RPC), :],
                             tbufs.at[slot], sems[slot])

        def wait(slot):
            pltpu.make_async_copy(in_hbm.at[0, :, pl.ds(0, RPC), :],
                                  xbufs.at[slot], sems[slot]).wait()
            pltpu.make_async_copy(tg_hbm.at[0, pl.ds(0, RPC), :],
                                  tbufs.at[slot], sems[slot]).wait()

        def group(slot, r, col):
            items = [(xbufs[slot, c, r, pl.ds(col, L)], c) for c in range(C)]
            while len(items) > 1:
                nxt = []
                for j in range(0, len(items) - 1, 2):
                    pm, pa = items[j]
                    qm, qa = items[j + 1]
                    gt = qm > pm
                    nxt.append((jnp.where(gt, qm, pm), jnp.where(gt, qa, pa)))
                if len(items) % 2:
                    nxt.append(items[-1])
                items = nxt
            a = items[0][1]
            t = tbufs[slot, r, pl.ds(col, L)]
            addr = lane_off + t * ROW + a
            plsc.addupdate_scatter(hist, [addr], ones)

        def compute(slot):
            for r in range(RPC):
                @plsc.parallel_loop(0, W // L, unroll=4)
                def _(i):
                    group(slot, r, i * L)

        issue(0, 0)
        issue(1, 1)

        def pair_body(cp, _):
            ci = cp * 2
            wait(0)
            compute(0)

            @pl.when(ci + 2 < total)
            def _():
                issue(ci + 2, 0)

            wait(1)
            compute(1)

            @pl.when(ci + 3 < total)
            def _():
                issue(ci + 3, 1)

            return 0

        lax.fori_loop(0, total // 2, pair_body, 0)
        pltpu.sync_copy(hist, out_hbm.at[wid])

    return k(inp, tgt)


def _merge(parts, C):
    def body(x_ref, o_ref):
        o_ref[...] = jnp.sum(x_ref[...], axis=(0, 1))[:, :C]

    return pl.pallas_call(
        body,
        out_shape=jax.ShapeDtypeStruct((C, C), jnp.int32),
    )(parts)


def kernel(input, target, class_num):
    C = input.shape[1]
    parts = _sc_partial_hist(input, target, C)
    parts = parts.reshape(NW, L, C, ROW)
    return _merge(parts, C)
